# Initial kernel scaffold; baseline (speedup 1.0000x reference)
import jax
import jax.numpy as jnp
from jax.experimental import pallas as pl


def kernel(x, table):
    raise NotImplementedError("write your pallas kernel here")


# speedup vs baseline: 1.0947x; 1.0947x over previous
import functools

import jax
import jax.numpy as jnp
from jax import lax
from jax.experimental import pallas as pl
from jax.experimental.pallas import tpu as pltpu
from jax.experimental.pallas import tpu_sc as plsc

EMBEDDING_DIM = 32
CHUNK = 1024


@functools.partial(jax.jit, static_argnames=())
def _embedding_lookup(idx_flat, table):
    info = plsc.get_sparse_core_info()
    num_workers = info.num_cores * info.num_subcores
    b_total = idx_flat.shape[0]
    b_per_w = b_total // num_workers
    n_chunks = b_per_w // CHUNK

    mesh = plsc.VectorSubcoreMesh(core_axis_name="c", subcore_axis_name="s")

    @functools.partial(
        pl.kernel,
        mesh=mesh,
        out_type=jax.ShapeDtypeStruct((b_total, EMBEDDING_DIM), jnp.float32),
        scratch_types=[
            pltpu.VMEM((CHUNK,), jnp.int32),
            pltpu.VMEM((CHUNK, EMBEDDING_DIM), jnp.float32),
            pltpu.SemaphoreType.DMA,
        ],
        compiler_params=pltpu.CompilerParams(use_tc_tiling_on_sc=False),
    )
    def emb_kernel(idx_hbm, table_hbm, out_hbm, idx_v, rows_v, sem):
        wid = lax.axis_index("s") * info.num_cores + lax.axis_index("c")
        base = wid * b_per_w

        def body(i, _):
            off = base + i * CHUNK
            pltpu.sync_copy(idx_hbm.at[pl.ds(off, CHUNK)], idx_v)
            pltpu.async_copy(table_hbm.at[idx_v], rows_v, sem).wait()
            pltpu.sync_copy(rows_v, out_hbm.at[pl.ds(off, CHUNK)])
            return 0

        lax.fori_loop(0, n_chunks, body, 0)

    return emb_kernel(idx_flat, table)


def kernel(x, table):
    idx_flat = x.reshape(-1).astype(jnp.int32)
    out = _embedding_lookup(idx_flat, table)
    return out.reshape(x.shape + (EMBEDDING_DIM,))


# speedup vs baseline: 1.1134x; 1.0171x over previous
import functools

import jax
import jax.numpy as jnp
from jax import lax
from jax.experimental import pallas as pl
from jax.experimental.pallas import tpu as pltpu
from jax.experimental.pallas import tpu_sc as plsc

EMBEDDING_DIM = 32
CHUNK = 1280
NBUF = 3


@jax.jit
def _embedding_lookup(idx_flat, table):
    info = plsc.get_sparse_core_info()
    num_workers = info.num_cores * info.num_subcores
    b_total = idx_flat.shape[0]
    b_per_w = b_total // num_workers
    n_chunks = b_per_w // CHUNK

    mesh = plsc.VectorSubcoreMesh(core_axis_name="c", subcore_axis_name="s")

    @functools.partial(
        pl.kernel,
        mesh=mesh,
        out_type=jax.ShapeDtypeStruct((b_total, EMBEDDING_DIM), jnp.float32),
        scratch_types=[
            pltpu.VMEM((NBUF, CHUNK), jnp.int32),
            pltpu.VMEM((NBUF, CHUNK, EMBEDDING_DIM), jnp.float32),
        ]
        + [pltpu.SemaphoreType.DMA] * (3 * NBUF),
        compiler_params=pltpu.CompilerParams(use_tc_tiling_on_sc=False),
    )
    def emb_kernel(idx_hbm, table_hbm, out_hbm, idx_v, rows_v, *sems):
        isem = sems[:NBUF]
        gsem = sems[NBUF : 2 * NBUF]
        ssem = sems[2 * NBUF :]
        wid = lax.axis_index("s") * info.num_cores + lax.axis_index("c")
        base = wid * b_per_w

        def idx_start(g, b):
            pltpu.async_copy(
                idx_hbm.at[pl.ds(base + g * CHUNK, CHUNK)], idx_v.at[b], isem[b]
            )

        def gather_start(g, b):
            del g
            pltpu.async_copy(table_hbm.at[idx_v.at[b]], rows_v.at[b], gsem[b])

        def gather_wait(b):
            pltpu.make_async_copy(
                table_hbm.at[idx_v.at[b]], rows_v.at[b], gsem[b]
            ).wait()

        def store_start(g, b):
            pltpu.async_copy(
                rows_v.at[b], out_hbm.at[pl.ds(base + g * CHUNK, CHUNK)], ssem[b]
            )

        def store_wait(g, b):
            pltpu.make_async_copy(
                rows_v.at[b], out_hbm.at[pl.ds(base + g * CHUNK, CHUNK)], ssem[b]
            ).wait()

        for b in range(NBUF):
            idx_start(b, b)
        pltpu.make_async_copy(
            idx_hbm.at[pl.ds(base, CHUNK)], idx_v.at[0], isem[0]
        ).wait()
        gather_start(0, 0)

        for g in range(n_chunks):
            b = g % NBUF
            gnext = g + 1
            if gnext < n_chunks:
                bn = gnext % NBUF
                pltpu.make_async_copy(
                    idx_hbm.at[pl.ds(base + gnext * CHUNK, CHUNK)],
                    idx_v.at[bn],
                    isem[bn],
                ).wait()
                if gnext >= NBUF:
                    store_wait(gnext - NBUF, bn)
                gather_start(gnext, bn)
            gather_wait(b)
            store_start(g, b)
            if g + NBUF < n_chunks:
                idx_start(g + NBUF, b)

        for g in range(n_chunks - NBUF, n_chunks):
            store_wait(g, g % NBUF)

    return emb_kernel(idx_flat, table)


def kernel(x, table):
    idx_flat = x.reshape(-1).astype(jnp.int32)
    out = _embedding_lookup(idx_flat, table)
    return out.reshape(x.shape + (EMBEDDING_DIM,))
